# baseline (speedup 1.0000x reference)
import functools
import jax
import jax.numpy as jnp
from jax import lax
from jax.experimental import pallas as pl
from jax.experimental.pallas import tpu as pltpu
from jax.experimental.pallas import tpu_sc as plsc

EMBED = 64
PAD = 128
SEQ = 200
CHUNKS = ((0, 128), (128, 72))
LANES = 16
NUM_WORKERS = 32
def kernel(inputs, word_table, pos_table):
    batch, seq = inputs.shape
    rows_per_w = batch // NUM_WORKERS
    word_padded = jnp.pad(word_table, ((0, 0), (0, PAD - EMBED)))

    mesh = plsc.VectorSubcoreMesh(core_axis_name="c", subcore_axis_name="s")

    @functools.partial(
        pl.kernel,
        out_type=jax.ShapeDtypeStruct((batch, SEQ, EMBED), jnp.float32),
        mesh=mesh,
        scratch_types=[
            pltpu.VMEM((rows_per_w, SEQ), jnp.int32),
            pltpu.VMEM((SEQ, EMBED), jnp.float32),
            pltpu.VMEM((CHUNKS[0][1], PAD), jnp.float32),
            pltpu.VMEM((SEQ, EMBED), jnp.float32),
        ],
    )
    def sc_kernel(word_hbm, idx_hbm, pos_hbm, out_hbm,
                  idx_v, pos_v, rows_v, stage_v):
        wid = lax.axis_index("s") * 2 + lax.axis_index("c")
        row_base = pl.multiple_of(wid * rows_per_w, rows_per_w)
        pltpu.sync_copy(idx_hbm.at[pl.ds(row_base, rows_per_w)], idx_v)
        pltpu.sync_copy(pos_hbm, pos_v)

        @pl.loop(0, rows_per_w)
        def _(t):
            for start, size in CHUNKS:
                pltpu.sync_copy(
                    word_hbm.at[idx_v.at[t, pl.ds(start, size)]],
                    rows_v.at[pl.ds(0, size)],
                )

                @pl.loop(0, size)
                def _(r):
                    for c in range(0, EMBED, LANES):
                        stage_v.at[start + r, pl.ds(c, LANES)][...] = (
                            rows_v.at[r, pl.ds(c, LANES)][...]
                            + pos_v.at[start + r, pl.ds(c, LANES)][...]
                        )

            pltpu.sync_copy(stage_v, out_hbm.at[row_base + t])

    return sc_kernel(word_padded, inputs, pos_table)


# speedup vs baseline: 1.5935x; 1.5935x over previous
import functools
import jax
import jax.numpy as jnp
from jax import lax
from jax.experimental import pallas as pl
from jax.experimental.pallas import tpu as pltpu
from jax.experimental.pallas import tpu_sc as plsc

EMBED = 64
PAD = 128
SEQ = 200
CHUNKS = ((0, 128), (128, 72))
LANES = 16
UNROLL = 4
NUM_WORKERS = 32


def kernel(inputs, word_table, pos_table):
    batch, seq = inputs.shape
    num_idx = batch * seq
    rows_per_w = batch // NUM_WORKERS
    idx_per_w = rows_per_w * seq
    flat_idx = inputs.reshape(num_idx)
    word_padded = jnp.pad(word_table, ((0, 0), (0, PAD - EMBED)))

    mesh = plsc.VectorSubcoreMesh(core_axis_name="c", subcore_axis_name="s")

    @functools.partial(
        pl.kernel,
        out_type=jax.ShapeDtypeStruct((num_idx, EMBED), jnp.float32),
        mesh=mesh,
        scratch_types=[
            pltpu.VMEM((idx_per_w,), jnp.int32),
            pltpu.VMEM((SEQ, EMBED), jnp.float32),
            pltpu.VMEM((CHUNKS[0][1], PAD), jnp.float32),
            pltpu.VMEM((CHUNKS[1][1], PAD), jnp.float32),
            pltpu.VMEM((2, SEQ, EMBED), jnp.float32),
            pltpu.SemaphoreType.DMA,
            pltpu.SemaphoreType.DMA,
            pltpu.SemaphoreType.DMA,
            pltpu.SemaphoreType.DMA,
        ],
    )
    def sc_kernel(word_hbm, idx_hbm, pos_hbm, out_hbm,
                  idx_v, pos_v, rows0_v, rows1_v, stage_v,
                  gsem0, gsem1, osem0, osem1):
        wid = lax.axis_index("s") * 2 + lax.axis_index("c")
        row_base = pl.multiple_of(wid * rows_per_w, rows_per_w)
        idx_base = pl.multiple_of(wid * idx_per_w, idx_per_w)
        pltpu.sync_copy(idx_hbm.at[pl.ds(idx_base, idx_per_w)], idx_v)
        pltpu.sync_copy(pos_hbm, pos_v)

        rows_bufs = (rows0_v, rows1_v)
        gsems = (gsem0, gsem1)
        osems = (osem0, osem1)

        def gather(t, h):
            start, size = CHUNKS[h]
            return pltpu.make_async_copy(
                word_hbm.at[idx_v.at[pl.ds(t * SEQ + start, size)]],
                rows_bufs[h], gsems[h],
            )

        def out_copy(t, slot):
            out_base = pl.multiple_of((row_base + t) * SEQ, SEQ)
            return pltpu.make_async_copy(
                stage_v.at[slot], out_hbm.at[pl.ds(out_base, SEQ)],
                osems[slot],
            )

        def do_row(t, slot, wait_out, issue_next):
            if wait_out:
                out_copy(t - 2, slot).wait()
            for h in range(2):
                start, size = CHUNKS[h]
                gather(t, h).wait()

                @pl.loop(0, size // UNROLL)
                def _(i):
                    r0 = i * UNROLL
                    for rr in range(UNROLL):
                        r = r0 + rr
                        for c in range(0, EMBED, LANES):
                            stage_v.at[slot, start + r, pl.ds(c, LANES)][...] = (
                                rows_bufs[h].at[r, pl.ds(c, LANES)][...]
                                + pos_v.at[start + r, pl.ds(c, LANES)][...]
                            )
                if issue_next:
                    gather(t + 1, h).start()
            out_copy(t, slot).start()

        gather(0, 0).start()
        gather(0, 1).start()
        do_row(0, 0, wait_out=False, issue_next=True)
        do_row(1, 1, wait_out=False, issue_next=True)

        @pl.loop(1, rows_per_w // 2 - 1)
        def _(u):
            do_row(2 * u, 0, wait_out=True, issue_next=True)
            do_row(2 * u + 1, 1, wait_out=True, issue_next=True)

        do_row(rows_per_w - 2, 0, wait_out=True, issue_next=True)
        do_row(rows_per_w - 1, 1, wait_out=True, issue_next=False)
        out_copy(rows_per_w - 2, 0).wait()
        out_copy(rows_per_w - 1, 1).wait()

    out = sc_kernel(word_padded, flat_idx, pos_table)
    return out.reshape(batch, seq, EMBED)
